# baseline (speedup 1.0000x reference)
import functools
import math

import jax
import jax.numpy as jnp
from jax.experimental import pallas as pl

NH = 16


def _qkv_kernel(x_ref, w_ref, b_ref, o_ref, *, heads_per_step, hs):
    r = jnp.dot(x_ref[...], w_ref[...], preferred_element_type=jnp.float32)
    r = r + b_ref[...]
    for hh in range(heads_per_step):
        o_ref[0, hh] = r[:, hh * hs:(hh + 1) * hs]


def _attn_kernel(q_ref, k_ref, v_ref, o_ref, *, bq, bk, scale):
    qi = pl.program_id(2)
    q = q_ref[0, 0]
    hs = q.shape[-1]

    m0 = jnp.full((bq, 1), -1e30, dtype=jnp.float32)
    l0 = jnp.zeros((bq, 1), dtype=jnp.float32)
    a0 = jnp.zeros((bq, hs), dtype=jnp.float32)

    row_ids = qi * bq + jax.lax.broadcasted_iota(jnp.int32, (bq, bk), 0)

    def step(j, carry):
        m, l, acc = carry
        kj = k_ref[0, 0, pl.ds(j * bk, bk), :]
        vj = v_ref[0, 0, pl.ds(j * bk, bk), :]
        s = jax.lax.dot_general(q, kj, (((1,), (1,)), ((), ())),
                                preferred_element_type=jnp.float32)
        s = s * scale
        col_ids = j * bk + jax.lax.broadcasted_iota(jnp.int32, (bq, bk), 1)
        s = jnp.where(col_ids <= row_ids, s, -1e30)
        m_new = jnp.maximum(m, jnp.max(s, axis=1, keepdims=True))
        p = jnp.exp(s - m_new)
        alpha = jnp.exp(m - m_new)
        l_new = l * alpha + jnp.sum(p, axis=1, keepdims=True)
        acc_new = acc * alpha + jnp.dot(p, vj, preferred_element_type=jnp.float32)
        return m_new, l_new, acc_new

    nblk = ((qi + 1) * bq) // bk
    m, l, acc = jax.lax.fori_loop(0, nblk, step, (m0, l0, a0))
    o_ref[0, 0] = acc / l


def _proj_kernel(y_ref, w_ref, b_ref, o_ref, *, nh, hs):
    acc = jnp.zeros((y_ref.shape[2], w_ref.shape[2]), dtype=jnp.float32)
    for h in range(nh):
        acc = acc + jnp.dot(y_ref[0, h], w_ref[h],
                            preferred_element_type=jnp.float32)
    o_ref[0] = acc + b_ref[...]


def kernel(x, W_attn, b_attn, W_proj, b_proj):
    B, T, C = x.shape
    HS = C // NH
    G = 3 * NH

    x2 = x.reshape(B * T, C)

    heads_per_step = 4
    bn1 = heads_per_step * HS
    ng1 = G // heads_per_step
    qkv = pl.pallas_call(
        functools.partial(_qkv_kernel, heads_per_step=heads_per_step, hs=HS),
        grid=(B, ng1),
        in_specs=[
            pl.BlockSpec((T, C), lambda b, j: (b, 0)),
            pl.BlockSpec((C, bn1), lambda b, j: (0, j)),
            pl.BlockSpec((1, bn1), lambda b, j: (0, j)),
        ],
        out_specs=pl.BlockSpec((1, heads_per_step, T, HS),
                               lambda b, j: (b, j, 0, 0)),
        out_shape=jax.ShapeDtypeStruct((B, G, T, HS), jnp.float32),
    )(x2, W_attn, b_attn.reshape(1, 3 * C))

    BQ = 256
    BK = 256
    nq = T // BQ
    scale = 1.0 / math.sqrt(HS)
    y = pl.pallas_call(
        functools.partial(_attn_kernel, bq=BQ, bk=BK, scale=scale),
        grid=(B, NH, nq),
        in_specs=[
            pl.BlockSpec((1, 1, BQ, HS), lambda b, h, qi: (b, h, qi, 0)),
            pl.BlockSpec((1, 1, T, HS), lambda b, h, qi: (b, NH + h, 0, 0)),
            pl.BlockSpec((1, 1, T, HS), lambda b, h, qi: (b, 2 * NH + h, 0, 0)),
        ],
        out_specs=pl.BlockSpec((1, 1, BQ, HS), lambda b, h, qi: (b, h, qi, 0)),
        out_shape=jax.ShapeDtypeStruct((B, NH, T, HS), jnp.float32),
    )(qkv, qkv, qkv)

    bn3 = 512
    nn3 = C // bn3
    out = pl.pallas_call(
        functools.partial(_proj_kernel, nh=NH, hs=HS),
        grid=(B, nn3),
        in_specs=[
            pl.BlockSpec((1, NH, T, HS), lambda b, j: (b, 0, 0, 0)),
            pl.BlockSpec((NH, HS, bn3), lambda b, j: (0, 0, j)),
            pl.BlockSpec((1, bn3), lambda b, j: (0, j)),
        ],
        out_specs=pl.BlockSpec((1, T, bn3), lambda b, j: (b, 0, j)),
        out_shape=jax.ShapeDtypeStruct((B, T, C), jnp.float32),
    )(y, W_proj.reshape(NH, HS, C), b_proj.reshape(1, C))

    return out
